# Initial kernel scaffold; baseline (speedup 1.0000x reference)
import jax
import jax.numpy as jnp
from jax.experimental import pallas as pl


def kernel(L_x_, L_edge_index_, L_self_modules_edge_lin_parameters_weight_, L_self_modules_edge_lin_parameters_bias_, L_self_modules_cat_lin1_parameters_weight_, L_self_modules_cat_lin1_parameters_bias_, L_self_modules_cat_lin2_parameters_weight_, L_self_modules_cat_lin2_parameters_bias_, L_self_modules_node_mlp_modules_lins_modules_0_parameters_weight_, L_self_modules_node_mlp_modules_lins_modules_0_parameters_bias_, L_self_modules_final_mlp_modules_lins_modules_0_parameters_weight_, L_self_modules_final_mlp_modules_lins_modules_0_parameters_bias_, L_self_modules_final_mlp_modules_lins_modules_1_parameters_weight_, L_self_modules_final_mlp_modules_lins_modules_1_parameters_bias_, L_self_modules_final_mlp_modules_lins_modules_2_parameters_weight_, L_self_modules_final_mlp_modules_lins_modules_2_parameters_bias_, L_self_modules_final_mlp_modules_lins_modules_3_parameters_weight_, L_self_modules_final_mlp_modules_lins_modules_3_parameters_bias_, L_self_modules_final_mlp_modules_lins_modules_4_parameters_weight_, L_self_modules_final_mlp_modules_lins_modules_4_parameters_bias_, L_self_modules_final_mlp_modules_norms_modules_0_modules_module_buffers_running_mean_, L_self_modules_final_mlp_modules_norms_modules_0_modules_module_buffers_running_var_, L_self_modules_final_mlp_modules_norms_modules_0_modules_module_parameters_weight_, L_self_modules_final_mlp_modules_norms_modules_0_modules_module_parameters_bias_, L_self_modules_final_mlp_modules_norms_modules_1_modules_module_buffers_running_mean_, L_self_modules_final_mlp_modules_norms_modules_1_modules_module_buffers_running_var_, L_self_modules_final_mlp_modules_norms_modules_1_modules_module_parameters_weight_, L_self_modules_final_mlp_modules_norms_modules_1_modules_module_parameters_bias_, L_self_modules_final_mlp_modules_norms_modules_2_modules_module_buffers_running_mean_, L_self_modules_final_mlp_modules_norms_modules_2_modules_module_buffers_running_var_, L_self_modules_final_mlp_modules_norms_modules_2_modules_module_parameters_weight_, L_self_modules_final_mlp_modules_norms_modules_2_modules_module_parameters_bias_, L_self_modules_final_mlp_modules_norms_modules_3_modules_module_buffers_running_mean_, L_self_modules_final_mlp_modules_norms_modules_3_modules_module_buffers_running_var_, L_self_modules_final_mlp_modules_norms_modules_3_modules_module_parameters_weight_, L_self_modules_final_mlp_modules_norms_modules_3_modules_module_parameters_bias_):
    raise NotImplementedError("write your pallas kernel here")


# speedup vs baseline: 4.2459x; 4.2459x over previous
import functools

import jax
import jax.numpy as jnp
from jax import lax
from jax.experimental import pallas as pl
from jax.experimental.pallas import tpu as pltpu
from jax.experimental.pallas import tpu_sc as plsc

N = 10000
E = 320000
D = 128

NC = 2
NS = 16
NW = NC * NS
EPW = E // NW
K = 80
NCHUNK = EPW // K
RPT = 624
TAIL = N - NS * RPT

BM = 1000
GRID = N // BM


def _pre_body(x_ref, wes_ref, wed_ref, be_ref, w1_ref, b1_ref,
              a_ref, b_ref, t0_ref):
    x = x_ref[...]
    hi = jax.lax.Precision.HIGHEST
    a_ref[...] = jnp.dot(x, wes_ref[...], precision=hi,
                         preferred_element_type=jnp.float32)
    b_ref[...] = jnp.dot(x, wed_ref[...], precision=hi,
                         preferred_element_type=jnp.float32) + be_ref[...]
    t0_ref[...] = jnp.dot(x, w1_ref[...], precision=hi,
                          preferred_element_type=jnp.float32) + b1_ref[...]


def _pre(x, wes_t, wed_t, be, w1_t, b1):
    row = pl.BlockSpec((BM, D), lambda i: (i, 0))
    wspec = pl.BlockSpec((D, D), lambda i: (0, 0))
    bspec = pl.BlockSpec((1, D), lambda i: (0, 0))
    return pl.pallas_call(
        _pre_body,
        grid=(GRID,),
        in_specs=[row, wspec, wspec, bspec, wspec, bspec],
        out_specs=[row, row, row],
        out_shape=[jax.ShapeDtypeStruct((N, D), jnp.float32)] * 3,
    )(x, wes_t, wed_t, be.reshape(1, D), w1_t, b1.reshape(1, D))


def _edge_agg(src, dst, a_tab, b_tab):
    mesh = plsc.VectorSubcoreMesh(core_axis_name="c", subcore_axis_name="s")

    @functools.partial(
        pl.kernel,
        mesh=mesh,
        out_type=jax.ShapeDtypeStruct((NC, N, D), jnp.float32),
        scratch_types=[
            pltpu.VMEM((K,), jnp.int32),
            pltpu.VMEM((K,), jnp.int32),
            pltpu.VMEM((K, D), jnp.float32),
            pltpu.VMEM((K, D), jnp.float32),
            pltpu.VMEM_SHARED((N, D), jnp.float32),
            pltpu.SemaphoreType.DMA,
            pltpu.SemaphoreType.DMA,
        ],
    )
    def k(src_hbm, dst_hbm, a_hbm, b_hbm, out_hbm,
          si_v, di_v, a_v, b_v, agg_sh, sem_a, sem_b):
        cid = lax.axis_index("c")
        sid = lax.axis_index("s")
        wid = cid * NS + sid

        zero = jnp.zeros((16,), jnp.float32)

        def zrow(j, carry):
            for r in range(8):
                b_v[j, pl.ds(r * 16, 16)] = zero
            return carry

        lax.fori_loop(0, K, zrow, 0)

        base_row = sid * RPT
        for r in range(7):
            pltpu.sync_copy(b_v, agg_sh.at[pl.ds(base_row + r * K, K)])
        pltpu.sync_copy(b_v.at[pl.ds(0, RPT - 7 * K)],
                        agg_sh.at[pl.ds(base_row + 7 * K, RPT - 7 * K)])

        @pl.when(sid == NS - 1)
        def _():
            pltpu.sync_copy(b_v.at[pl.ds(0, TAIL)],
                            agg_sh.at[pl.ds(NS * RPT, TAIL)])

        plsc.subcore_barrier()

        ebase = wid * EPW

        def chunk(c, carry):
            off = ebase + c * K
            pltpu.sync_copy(src_hbm.at[pl.ds(off, K)], si_v)
            pltpu.sync_copy(dst_hbm.at[pl.ds(off, K)], di_v)
            cp_a = pltpu.async_copy(a_hbm.at[si_v], a_v, sem_a)
            cp_b = pltpu.async_copy(b_hbm.at[di_v], b_v, sem_b)
            cp_a.wait()
            cp_b.wait()

            def row(j, cc):
                for r in range(8):
                    s = pl.ds(r * 16, 16)
                    b_v[j, s] = jnp.maximum(a_v[j, s] + b_v[j, s], 0.0)
                return cc

            lax.fori_loop(0, K, row, 0)
            pltpu.sync_copy(b_v, agg_sh.at[di_v], add=True)
            return carry

        lax.fori_loop(0, NCHUNK, chunk, 0)
        plsc.subcore_barrier()
        pltpu.sync_copy(agg_sh.at[pl.ds(base_row, RPT)],
                        out_hbm.at[cid, pl.ds(base_row, RPT)])

        @pl.when(sid == NS - 1)
        def _():
            pltpu.sync_copy(agg_sh.at[pl.ds(NS * RPT, TAIL)],
                            out_hbm.at[cid, pl.ds(NS * RPT, TAIL)])

    return k(src, dst, a_tab, b_tab)


def _tail_body(agg_ref, t0_ref, wn_ref, bn_ref, w2_ref, b2_ref,
               wl_ref, bl_ref, rm_ref, rv_ref, g_ref, bb_ref,
               w4_ref, b4_ref, out_ref):
    hi = jax.lax.Precision.HIGHEST
    agg = agg_ref[0] + agg_ref[1]
    h = jnp.maximum(
        jnp.dot(agg, wn_ref[...], precision=hi,
                preferred_element_type=jnp.float32) + bn_ref[...], 0.0)
    t = t0_ref[...] + jnp.dot(h, w2_ref[...], precision=hi,
                              preferred_element_type=jnp.float32) + b2_ref[...]
    for j in range(4):
        z = jnp.dot(t, wl_ref[j], precision=hi,
                    preferred_element_type=jnp.float32) + bl_ref[j]
        scale = jax.lax.rsqrt(rv_ref[j] + 1e-5) * g_ref[j]
        t = jnp.maximum((z - rm_ref[j]) * scale + bb_ref[j], 0.0)
    out_ref[...] = jnp.dot(t, w4_ref[...], precision=hi,
                           preferred_element_type=jnp.float32) + b4_ref[...]


def _tail(aggp, t0, wn_t, bn, w2_t, b2, wl_t, bl, rm, rv, g, bb, w4_t, b4):
    row = pl.BlockSpec((BM, D), lambda i: (i, 0))
    aggspec = pl.BlockSpec((NC, BM, D), lambda i: (0, i, 0))
    wspec = pl.BlockSpec((D, D), lambda i: (0, 0))
    bspec = pl.BlockSpec((1, D), lambda i: (0, 0))
    wlspec = pl.BlockSpec((4, D, D), lambda i: (0, 0, 0))
    blspec = pl.BlockSpec((4, 1, D), lambda i: (0, 0, 0))
    return pl.pallas_call(
        _tail_body,
        grid=(GRID,),
        in_specs=[aggspec, row, wspec, bspec, wspec, bspec,
                  wlspec, blspec, blspec, blspec, blspec, blspec,
                  wspec, bspec],
        out_specs=row,
        out_shape=jax.ShapeDtypeStruct((N, D), jnp.float32),
    )(aggp, t0, wn_t, bn.reshape(1, D), w2_t, b2.reshape(1, D),
      wl_t, bl.reshape(4, 1, D), rm.reshape(4, 1, D), rv.reshape(4, 1, D),
      g.reshape(4, 1, D), bb.reshape(4, 1, D), w4_t, b4.reshape(1, D))


def kernel(L_x_, L_edge_index_, L_self_modules_edge_lin_parameters_weight_, L_self_modules_edge_lin_parameters_bias_, L_self_modules_cat_lin1_parameters_weight_, L_self_modules_cat_lin1_parameters_bias_, L_self_modules_cat_lin2_parameters_weight_, L_self_modules_cat_lin2_parameters_bias_, L_self_modules_node_mlp_modules_lins_modules_0_parameters_weight_, L_self_modules_node_mlp_modules_lins_modules_0_parameters_bias_, L_self_modules_final_mlp_modules_lins_modules_0_parameters_weight_, L_self_modules_final_mlp_modules_lins_modules_0_parameters_bias_, L_self_modules_final_mlp_modules_lins_modules_1_parameters_weight_, L_self_modules_final_mlp_modules_lins_modules_1_parameters_bias_, L_self_modules_final_mlp_modules_lins_modules_2_parameters_weight_, L_self_modules_final_mlp_modules_lins_modules_2_parameters_bias_, L_self_modules_final_mlp_modules_lins_modules_3_parameters_weight_, L_self_modules_final_mlp_modules_lins_modules_3_parameters_bias_, L_self_modules_final_mlp_modules_lins_modules_4_parameters_weight_, L_self_modules_final_mlp_modules_lins_modules_4_parameters_bias_, L_self_modules_final_mlp_modules_norms_modules_0_modules_module_buffers_running_mean_, L_self_modules_final_mlp_modules_norms_modules_0_modules_module_buffers_running_var_, L_self_modules_final_mlp_modules_norms_modules_0_modules_module_parameters_weight_, L_self_modules_final_mlp_modules_norms_modules_0_modules_module_parameters_bias_, L_self_modules_final_mlp_modules_norms_modules_1_modules_module_buffers_running_mean_, L_self_modules_final_mlp_modules_norms_modules_1_modules_module_buffers_running_var_, L_self_modules_final_mlp_modules_norms_modules_1_modules_module_parameters_weight_, L_self_modules_final_mlp_modules_norms_modules_1_modules_module_parameters_bias_, L_self_modules_final_mlp_modules_norms_modules_2_modules_module_buffers_running_mean_, L_self_modules_final_mlp_modules_norms_modules_2_modules_module_buffers_running_var_, L_self_modules_final_mlp_modules_norms_modules_2_modules_module_parameters_weight_, L_self_modules_final_mlp_modules_norms_modules_2_modules_module_parameters_bias_, L_self_modules_final_mlp_modules_norms_modules_3_modules_module_buffers_running_mean_, L_self_modules_final_mlp_modules_norms_modules_3_modules_module_buffers_running_var_, L_self_modules_final_mlp_modules_norms_modules_3_modules_module_parameters_weight_, L_self_modules_final_mlp_modules_norms_modules_3_modules_module_parameters_bias_):
    x = L_x_
    ei = L_edge_index_
    src = ei[0].astype(jnp.int32)
    dst = ei[1].astype(jnp.int32)
    we = L_self_modules_edge_lin_parameters_weight_
    be = L_self_modules_edge_lin_parameters_bias_

    a_tab, b_tab, t0 = _pre(
        x, we[:, :D].T, we[:, D:].T, be,
        L_self_modules_cat_lin1_parameters_weight_.T,
        L_self_modules_cat_lin1_parameters_bias_)

    aggp = _edge_agg(src, dst, a_tab, b_tab)

    wl_t = jnp.stack([
        L_self_modules_final_mlp_modules_lins_modules_0_parameters_weight_.T,
        L_self_modules_final_mlp_modules_lins_modules_1_parameters_weight_.T,
        L_self_modules_final_mlp_modules_lins_modules_2_parameters_weight_.T,
        L_self_modules_final_mlp_modules_lins_modules_3_parameters_weight_.T,
    ])
    bl = jnp.stack([
        L_self_modules_final_mlp_modules_lins_modules_0_parameters_bias_,
        L_self_modules_final_mlp_modules_lins_modules_1_parameters_bias_,
        L_self_modules_final_mlp_modules_lins_modules_2_parameters_bias_,
        L_self_modules_final_mlp_modules_lins_modules_3_parameters_bias_,
    ])
    rm = jnp.stack([
        L_self_modules_final_mlp_modules_norms_modules_0_modules_module_buffers_running_mean_,
        L_self_modules_final_mlp_modules_norms_modules_1_modules_module_buffers_running_mean_,
        L_self_modules_final_mlp_modules_norms_modules_2_modules_module_buffers_running_mean_,
        L_self_modules_final_mlp_modules_norms_modules_3_modules_module_buffers_running_mean_,
    ])
    rv = jnp.stack([
        L_self_modules_final_mlp_modules_norms_modules_0_modules_module_buffers_running_var_,
        L_self_modules_final_mlp_modules_norms_modules_1_modules_module_buffers_running_var_,
        L_self_modules_final_mlp_modules_norms_modules_2_modules_module_buffers_running_var_,
        L_self_modules_final_mlp_modules_norms_modules_3_modules_module_buffers_running_var_,
    ])
    g = jnp.stack([
        L_self_modules_final_mlp_modules_norms_modules_0_modules_module_parameters_weight_,
        L_self_modules_final_mlp_modules_norms_modules_1_modules_module_parameters_weight_,
        L_self_modules_final_mlp_modules_norms_modules_2_modules_module_parameters_weight_,
        L_self_modules_final_mlp_modules_norms_modules_3_modules_module_parameters_weight_,
    ])
    bb = jnp.stack([
        L_self_modules_final_mlp_modules_norms_modules_0_modules_module_parameters_bias_,
        L_self_modules_final_mlp_modules_norms_modules_1_modules_module_parameters_bias_,
        L_self_modules_final_mlp_modules_norms_modules_2_modules_module_parameters_bias_,
        L_self_modules_final_mlp_modules_norms_modules_3_modules_module_parameters_bias_,
    ])

    return _tail(
        aggp, t0,
        L_self_modules_node_mlp_modules_lins_modules_0_parameters_weight_.T,
        L_self_modules_node_mlp_modules_lins_modules_0_parameters_bias_,
        L_self_modules_cat_lin2_parameters_weight_.T,
        L_self_modules_cat_lin2_parameters_bias_,
        wl_t, bl, rm, rv, g, bb,
        L_self_modules_final_mlp_modules_lins_modules_4_parameters_weight_.T,
        L_self_modules_final_mlp_modules_lins_modules_4_parameters_bias_)


# speedup vs baseline: 5.7352x; 1.3508x over previous
import functools

import jax
import jax.numpy as jnp
from jax import lax
from jax.experimental import pallas as pl
from jax.experimental.pallas import tpu as pltpu
from jax.experimental.pallas import tpu_sc as plsc

N = 10000
E = 320000
D = 128

NC = 2
NS = 16
NW = NC * NS
EPW = E // NW
K = 80
NCHUNK = EPW // K
RPT = 624
TAIL = N - NS * RPT

BM = 1000
GRID = N // BM


def _pre_body(x_ref, wes_ref, wed_ref, be_ref, w1_ref, b1_ref,
              a_ref, b_ref, t0_ref):
    x = x_ref[...]
    hi = jax.lax.Precision.HIGHEST
    a_ref[...] = jnp.dot(x, wes_ref[...], precision=hi,
                         preferred_element_type=jnp.float32)
    b_ref[...] = jnp.dot(x, wed_ref[...], precision=hi,
                         preferred_element_type=jnp.float32) + be_ref[...]
    t0_ref[...] = jnp.dot(x, w1_ref[...], precision=hi,
                          preferred_element_type=jnp.float32) + b1_ref[...]


def _pre(x, wes_t, wed_t, be, w1_t, b1):
    row = pl.BlockSpec((BM, D), lambda i: (i, 0))
    wspec = pl.BlockSpec((D, D), lambda i: (0, 0))
    bspec = pl.BlockSpec((1, D), lambda i: (0, 0))
    return pl.pallas_call(
        _pre_body,
        grid=(GRID,),
        in_specs=[row, wspec, wspec, bspec, wspec, bspec],
        out_specs=[row, row, row],
        out_shape=[jax.ShapeDtypeStruct((N, D), jnp.float32)] * 3,
    )(x, wes_t, wed_t, be.reshape(1, D), w1_t, b1.reshape(1, D))


def _edge_agg(src, dst, a_tab, b_tab):
    mesh = plsc.VectorSubcoreMesh(core_axis_name="c", subcore_axis_name="s")

    @functools.partial(
        pl.kernel,
        mesh=mesh,
        out_type=jax.ShapeDtypeStruct((NC, N, D), jnp.float32),
        scratch_types=[
            pltpu.VMEM((K,), jnp.int32),
            pltpu.VMEM((K,), jnp.int32),
            pltpu.VMEM((K,), jnp.int32),
            pltpu.VMEM((K,), jnp.int32),
            pltpu.VMEM((K, D), jnp.float32),
            pltpu.VMEM((K, D), jnp.float32),
            pltpu.VMEM((K, D), jnp.float32),
            pltpu.VMEM((K, D), jnp.float32),
            pltpu.VMEM_SHARED((N, D), jnp.float32),
            pltpu.SemaphoreType.DMA,
            pltpu.SemaphoreType.DMA,
            pltpu.SemaphoreType.DMA,
            pltpu.SemaphoreType.DMA,
        ],
    )
    def k(src_hbm, dst_hbm, a_hbm, b_hbm, out_hbm,
          si0, si1, di0, di1, a0, a1, b0, b1, agg_sh,
          gsem0, gsem1, isem0, isem1):
        cid = lax.axis_index("c")
        sid = lax.axis_index("s")
        wid = cid * NS + sid
        sibuf = (si0, si1)
        dibuf = (di0, di1)
        abuf = (a0, a1)
        bbuf = (b0, b1)
        gsem = (gsem0, gsem1)
        isem = (isem0, isem1)
        ebase = wid * EPW

        zero = jnp.zeros((16,), jnp.float32)

        def zrow(j, carry):
            for r in range(8):
                b0[j, pl.ds(r * 16, 16)] = zero
            return carry

        lax.fori_loop(0, K, zrow, 0)

        base_row = sid * RPT
        for r in range(RPT // K):
            pltpu.sync_copy(b0, agg_sh.at[pl.ds(base_row + r * K, K)])
        if RPT % K:
            pltpu.sync_copy(
                b0.at[pl.ds(0, RPT % K)],
                agg_sh.at[pl.ds(base_row + (RPT // K) * K, RPT % K)])

        @pl.when(sid == NS - 1)
        def _():
            pltpu.sync_copy(b0.at[pl.ds(0, TAIL)],
                            agg_sh.at[pl.ds(NS * RPT, TAIL)])

        plsc.subcore_barrier()

        def fetch_idx(c, s):
            off = ebase + c * K
            pltpu.async_copy(src_hbm.at[pl.ds(off, K)], sibuf[s], isem[s])
            pltpu.async_copy(dst_hbm.at[pl.ds(off, K)], dibuf[s], isem[s])

        def wait_idx(s):
            pltpu.make_async_copy(src_hbm.at[pl.ds(0, K)], sibuf[s],
                                  isem[s]).wait()
            pltpu.make_async_copy(dst_hbm.at[pl.ds(0, K)], dibuf[s],
                                  isem[s]).wait()

        def gathers(s):
            pltpu.async_copy(a_hbm.at[sibuf[s]], abuf[s], gsem[s])
            pltpu.async_copy(b_hbm.at[dibuf[s]], bbuf[s], gsem[s])

        def wait_gathers(s):
            pltpu.make_async_copy(a_hbm.at[sibuf[s]], abuf[s],
                                  gsem[s]).wait()
            pltpu.make_async_copy(b_hbm.at[dibuf[s]], bbuf[s],
                                  gsem[s]).wait()

        def work(s):
            wait_gathers(s)
            a_v, b_v = abuf[s], bbuf[s]

            def row(j, cc):
                for r in range(8):
                    sl = pl.ds(r * 16, 16)
                    b_v[j, sl] = jnp.maximum(a_v[j, sl] + b_v[j, sl], 0.0)
                return cc

            lax.fori_loop(0, K, row, 0)
            pltpu.sync_copy(b_v, agg_sh.at[dibuf[s]], add=True)

        fetch_idx(0, 0)
        wait_idx(0)
        gathers(0)
        fetch_idx(1, 1)

        def pair(i, carry):
            c = 2 * i
            wait_idx(1)
            gathers(1)
            work(0)
            fetch_idx(c + 2, 0)
            work(1)
            wait_idx(0)
            gathers(0)
            fetch_idx(jnp.minimum(c + 3, NCHUNK - 1), 1)
            return carry

        lax.fori_loop(0, (NCHUNK - 1) // 2, pair, 0)
        work(0)
        wait_idx(1)
        plsc.subcore_barrier()
        pltpu.sync_copy(agg_sh.at[pl.ds(base_row, RPT)],
                        out_hbm.at[cid, pl.ds(base_row, RPT)])

        @pl.when(sid == NS - 1)
        def _():
            pltpu.sync_copy(agg_sh.at[pl.ds(NS * RPT, TAIL)],
                            out_hbm.at[cid, pl.ds(NS * RPT, TAIL)])

    return k(src, dst, a_tab, b_tab)


def _tail_body(agg_ref, t0_ref, wn_ref, bn_ref, w2_ref, b2_ref,
               wl_ref, bl_ref, rm_ref, rv_ref, g_ref, bb_ref,
               w4_ref, b4_ref, out_ref):
    hi = jax.lax.Precision.HIGHEST
    agg = agg_ref[0] + agg_ref[1]
    h = jnp.maximum(
        jnp.dot(agg, wn_ref[...], precision=hi,
                preferred_element_type=jnp.float32) + bn_ref[...], 0.0)
    t = t0_ref[...] + jnp.dot(h, w2_ref[...], precision=hi,
                              preferred_element_type=jnp.float32) + b2_ref[...]
    for j in range(4):
        z = jnp.dot(t, wl_ref[j], precision=hi,
                    preferred_element_type=jnp.float32) + bl_ref[j]
        scale = jax.lax.rsqrt(rv_ref[j] + 1e-5) * g_ref[j]
        t = jnp.maximum((z - rm_ref[j]) * scale + bb_ref[j], 0.0)
    out_ref[...] = jnp.dot(t, w4_ref[...], precision=hi,
                           preferred_element_type=jnp.float32) + b4_ref[...]


def _tail(aggp, t0, wn_t, bn, w2_t, b2, wl_t, bl, rm, rv, g, bb, w4_t, b4):
    row = pl.BlockSpec((BM, D), lambda i: (i, 0))
    aggspec = pl.BlockSpec((NC, BM, D), lambda i: (0, i, 0))
    wspec = pl.BlockSpec((D, D), lambda i: (0, 0))
    bspec = pl.BlockSpec((1, D), lambda i: (0, 0))
    wlspec = pl.BlockSpec((4, D, D), lambda i: (0, 0, 0))
    blspec = pl.BlockSpec((4, 1, D), lambda i: (0, 0, 0))
    return pl.pallas_call(
        _tail_body,
        grid=(GRID,),
        in_specs=[aggspec, row, wspec, bspec, wspec, bspec,
                  wlspec, blspec, blspec, blspec, blspec, blspec,
                  wspec, bspec],
        out_specs=row,
        out_shape=jax.ShapeDtypeStruct((N, D), jnp.float32),
    )(aggp, t0, wn_t, bn.reshape(1, D), w2_t, b2.reshape(1, D),
      wl_t, bl.reshape(4, 1, D), rm.reshape(4, 1, D), rv.reshape(4, 1, D),
      g.reshape(4, 1, D), bb.reshape(4, 1, D), w4_t, b4.reshape(1, D))


def kernel(L_x_, L_edge_index_, L_self_modules_edge_lin_parameters_weight_, L_self_modules_edge_lin_parameters_bias_, L_self_modules_cat_lin1_parameters_weight_, L_self_modules_cat_lin1_parameters_bias_, L_self_modules_cat_lin2_parameters_weight_, L_self_modules_cat_lin2_parameters_bias_, L_self_modules_node_mlp_modules_lins_modules_0_parameters_weight_, L_self_modules_node_mlp_modules_lins_modules_0_parameters_bias_, L_self_modules_final_mlp_modules_lins_modules_0_parameters_weight_, L_self_modules_final_mlp_modules_lins_modules_0_parameters_bias_, L_self_modules_final_mlp_modules_lins_modules_1_parameters_weight_, L_self_modules_final_mlp_modules_lins_modules_1_parameters_bias_, L_self_modules_final_mlp_modules_lins_modules_2_parameters_weight_, L_self_modules_final_mlp_modules_lins_modules_2_parameters_bias_, L_self_modules_final_mlp_modules_lins_modules_3_parameters_weight_, L_self_modules_final_mlp_modules_lins_modules_3_parameters_bias_, L_self_modules_final_mlp_modules_lins_modules_4_parameters_weight_, L_self_modules_final_mlp_modules_lins_modules_4_parameters_bias_, L_self_modules_final_mlp_modules_norms_modules_0_modules_module_buffers_running_mean_, L_self_modules_final_mlp_modules_norms_modules_0_modules_module_buffers_running_var_, L_self_modules_final_mlp_modules_norms_modules_0_modules_module_parameters_weight_, L_self_modules_final_mlp_modules_norms_modules_0_modules_module_parameters_bias_, L_self_modules_final_mlp_modules_norms_modules_1_modules_module_buffers_running_mean_, L_self_modules_final_mlp_modules_norms_modules_1_modules_module_buffers_running_var_, L_self_modules_final_mlp_modules_norms_modules_1_modules_module_parameters_weight_, L_self_modules_final_mlp_modules_norms_modules_1_modules_module_parameters_bias_, L_self_modules_final_mlp_modules_norms_modules_2_modules_module_buffers_running_mean_, L_self_modules_final_mlp_modules_norms_modules_2_modules_module_buffers_running_var_, L_self_modules_final_mlp_modules_norms_modules_2_modules_module_parameters_weight_, L_self_modules_final_mlp_modules_norms_modules_2_modules_module_parameters_bias_, L_self_modules_final_mlp_modules_norms_modules_3_modules_module_buffers_running_mean_, L_self_modules_final_mlp_modules_norms_modules_3_modules_module_buffers_running_var_, L_self_modules_final_mlp_modules_norms_modules_3_modules_module_parameters_weight_, L_self_modules_final_mlp_modules_norms_modules_3_modules_module_parameters_bias_):
    x = L_x_
    ei = L_edge_index_
    src = ei[0].astype(jnp.int32)
    dst = ei[1].astype(jnp.int32)
    we = L_self_modules_edge_lin_parameters_weight_
    be = L_self_modules_edge_lin_parameters_bias_

    a_tab, b_tab, t0 = _pre(
        x, we[:, :D].T, we[:, D:].T, be,
        L_self_modules_cat_lin1_parameters_weight_.T,
        L_self_modules_cat_lin1_parameters_bias_)

    aggp = _edge_agg(src, dst, a_tab, b_tab)

    wl_t = jnp.stack([
        L_self_modules_final_mlp_modules_lins_modules_0_parameters_weight_.T,
        L_self_modules_final_mlp_modules_lins_modules_1_parameters_weight_.T,
        L_self_modules_final_mlp_modules_lins_modules_2_parameters_weight_.T,
        L_self_modules_final_mlp_modules_lins_modules_3_parameters_weight_.T,
    ])
    bl = jnp.stack([
        L_self_modules_final_mlp_modules_lins_modules_0_parameters_bias_,
        L_self_modules_final_mlp_modules_lins_modules_1_parameters_bias_,
        L_self_modules_final_mlp_modules_lins_modules_2_parameters_bias_,
        L_self_modules_final_mlp_modules_lins_modules_3_parameters_bias_,
    ])
    rm = jnp.stack([
        L_self_modules_final_mlp_modules_norms_modules_0_modules_module_buffers_running_mean_,
        L_self_modules_final_mlp_modules_norms_modules_1_modules_module_buffers_running_mean_,
        L_self_modules_final_mlp_modules_norms_modules_2_modules_module_buffers_running_mean_,
        L_self_modules_final_mlp_modules_norms_modules_3_modules_module_buffers_running_mean_,
    ])
    rv = jnp.stack([
        L_self_modules_final_mlp_modules_norms_modules_0_modules_module_buffers_running_var_,
        L_self_modules_final_mlp_modules_norms_modules_1_modules_module_buffers_running_var_,
        L_self_modules_final_mlp_modules_norms_modules_2_modules_module_buffers_running_var_,
        L_self_modules_final_mlp_modules_norms_modules_3_modules_module_buffers_running_var_,
    ])
    g = jnp.stack([
        L_self_modules_final_mlp_modules_norms_modules_0_modules_module_parameters_weight_,
        L_self_modules_final_mlp_modules_norms_modules_1_modules_module_parameters_weight_,
        L_self_modules_final_mlp_modules_norms_modules_2_modules_module_parameters_weight_,
        L_self_modules_final_mlp_modules_norms_modules_3_modules_module_parameters_weight_,
    ])
    bb = jnp.stack([
        L_self_modules_final_mlp_modules_norms_modules_0_modules_module_parameters_bias_,
        L_self_modules_final_mlp_modules_norms_modules_1_modules_module_parameters_bias_,
        L_self_modules_final_mlp_modules_norms_modules_2_modules_module_parameters_bias_,
        L_self_modules_final_mlp_modules_norms_modules_3_modules_module_parameters_bias_,
    ])

    return _tail(
        aggp, t0,
        L_self_modules_node_mlp_modules_lins_modules_0_parameters_weight_.T,
        L_self_modules_node_mlp_modules_lins_modules_0_parameters_bias_,
        L_self_modules_cat_lin2_parameters_weight_.T,
        L_self_modules_cat_lin2_parameters_bias_,
        wl_t, bl, rm, rv, g, bb,
        L_self_modules_final_mlp_modules_lins_modules_4_parameters_weight_.T,
        L_self_modules_final_mlp_modules_lins_modules_4_parameters_bias_)


# speedup vs baseline: 6.4072x; 1.1172x over previous
import functools

import jax
import jax.numpy as jnp
from jax import lax
from jax.experimental import pallas as pl
from jax.experimental.pallas import tpu as pltpu
from jax.experimental.pallas import tpu_sc as plsc

N = 10000
E = 320000
D = 128

NC = 2
NS = 16
NW = NC * NS
EPW = E // NW
K = 80
NCHUNK = EPW // K
RPT = 624
TAIL = N - NS * RPT

BM = 1000
GRID = N // BM


def _pre_body(x_ref, wes_ref, wed_ref, be_ref, w1_ref, b1_ref,
              a_ref, b_ref, t0_ref):
    x = x_ref[...]
    hi = jax.lax.Precision.HIGHEST
    a_ref[...] = jnp.dot(x, wes_ref[...], precision=hi,
                         preferred_element_type=jnp.float32)
    b_ref[...] = jnp.dot(x, wed_ref[...], precision=hi,
                         preferred_element_type=jnp.float32) + be_ref[...]
    t0_ref[...] = jnp.dot(x, w1_ref[...], precision=hi,
                          preferred_element_type=jnp.float32) + b1_ref[...]


def _pre(x, wes_t, wed_t, be, w1_t, b1):
    row = pl.BlockSpec((BM, D), lambda i: (i, 0))
    wspec = pl.BlockSpec((D, D), lambda i: (0, 0))
    bspec = pl.BlockSpec((1, D), lambda i: (0, 0))
    return pl.pallas_call(
        _pre_body,
        grid=(GRID,),
        in_specs=[row, wspec, wspec, bspec, wspec, bspec],
        out_specs=[row, row, row],
        out_shape=[jax.ShapeDtypeStruct((N, D), jnp.float32)] * 3,
    )(x, wes_t, wed_t, be.reshape(1, D), w1_t, b1.reshape(1, D))


def _edge_agg(src, dst, a_tab, b_tab):
    mesh = plsc.VectorSubcoreMesh(core_axis_name="c", subcore_axis_name="s")

    @functools.partial(
        pl.kernel,
        mesh=mesh,
        out_type=jax.ShapeDtypeStruct((NC, N, D), jnp.float32),
        scratch_types=[
            pltpu.VMEM((K,), jnp.int32),
            pltpu.VMEM((K,), jnp.int32),
            pltpu.VMEM((K,), jnp.int32),
            pltpu.VMEM((K,), jnp.int32),
            pltpu.VMEM((K,), jnp.int32),
            pltpu.VMEM((K,), jnp.int32),
            pltpu.VMEM((K, D), jnp.float32),
            pltpu.VMEM((K, D), jnp.float32),
            pltpu.VMEM((K, D), jnp.float32),
            pltpu.VMEM((K, D), jnp.float32),
            pltpu.VMEM_SHARED((N, D), jnp.float32),
            pltpu.SemaphoreType.DMA,
            pltpu.SemaphoreType.DMA,
            pltpu.SemaphoreType.DMA,
            pltpu.SemaphoreType.DMA,
            pltpu.SemaphoreType.DMA,
            pltpu.SemaphoreType.DMA,
        ],
    )
    def k(src_hbm, dst_hbm, a_hbm, b_hbm, out_hbm,
          si0, si1, di0, di1, sd0, sd1, a0, a1, b0, b1, agg_sh,
          gsem0, gsem1, isem0, isem1, ssem0, ssem1):
        cid = lax.axis_index("c")
        sid = lax.axis_index("s")
        wid = cid * NS + sid
        sibuf = (si0, si1)
        dibuf = (di0, di1)
        sdbuf = (sd0, sd1)
        abuf = (a0, a1)
        bbuf = (b0, b1)
        gsem = (gsem0, gsem1)
        isem = (isem0, isem1)
        ssem = (ssem0, ssem1)
        ebase = wid * EPW

        zero = jnp.zeros((16,), jnp.float32)

        def zrow(j, carry):
            for r in range(8):
                b0[j, pl.ds(r * 16, 16)] = zero
            return carry

        lax.fori_loop(0, K, zrow, 0)

        base_row = sid * RPT
        for r in range(RPT // K):
            pltpu.sync_copy(b0, agg_sh.at[pl.ds(base_row + r * K, K)])
        if RPT % K:
            pltpu.sync_copy(
                b0.at[pl.ds(0, RPT % K)],
                agg_sh.at[pl.ds(base_row + (RPT // K) * K, RPT % K)])

        @pl.when(sid == NS - 1)
        def _():
            pltpu.sync_copy(b0.at[pl.ds(0, TAIL)],
                            agg_sh.at[pl.ds(NS * RPT, TAIL)])

        plsc.subcore_barrier()

        def fetch_idx(c, s):
            off = ebase + c * K
            pltpu.async_copy(src_hbm.at[pl.ds(off, K)], sibuf[s], isem[s])
            pltpu.async_copy(dst_hbm.at[pl.ds(off, K)], dibuf[s], isem[s])

        def wait_idx(s):
            pltpu.make_async_copy(src_hbm.at[pl.ds(0, K)], sibuf[s],
                                  isem[s]).wait()
            pltpu.make_async_copy(dst_hbm.at[pl.ds(0, K)], dibuf[s],
                                  isem[s]).wait()

        def gathers(s):
            pltpu.async_copy(a_hbm.at[sibuf[s]], abuf[s], gsem[s])
            pltpu.async_copy(b_hbm.at[dibuf[s]], bbuf[s], gsem[s])

        def wait_gathers(s):
            pltpu.make_async_copy(a_hbm.at[sibuf[s]], abuf[s],
                                  gsem[s]).wait()
            pltpu.make_async_copy(b_hbm.at[dibuf[s]], bbuf[s],
                                  gsem[s]).wait()

        def compute(s):
            wait_gathers(s)
            a_v, b_v = abuf[s], bbuf[s]

            def row(j, cc):
                for u in range(2):
                    for r in range(8):
                        sl = pl.ds(r * 16, 16)
                        b_v[2 * j + u, sl] = jnp.maximum(
                            a_v[2 * j + u, sl] + b_v[2 * j + u, sl], 0.0)
                return cc

            lax.fori_loop(0, K // 2, row, 0)
            for r in range(K // 16):
                sl = pl.ds(r * 16, 16)
                sdbuf[s][sl] = dibuf[s][sl]

        def scatter(s):
            pltpu.async_copy(bbuf[s], agg_sh.at[sdbuf[s]], ssem[s],
                             add=True)

        def wait_scatter(s):
            pltpu.make_async_copy(bbuf[s], agg_sh.at[sdbuf[s]],
                                  ssem[s]).wait()

        fetch_idx(0, 0)
        wait_idx(0)
        gathers(0)
        fetch_idx(1, 1)

        def pair(i, carry):
            c = 2 * i
            wait_idx(1)

            @pl.when(i > 0)
            def _():
                wait_scatter(1)

            gathers(1)
            compute(0)
            scatter(0)
            fetch_idx(c + 2, 0)
            compute(1)
            scatter(1)
            fetch_idx(jnp.minimum(c + 3, NCHUNK - 1), 1)
            wait_idx(0)
            wait_scatter(0)
            gathers(0)
            return carry

        lax.fori_loop(0, (NCHUNK - 1) // 2, pair, 0)
        compute(0)
        scatter(0)
        wait_scatter(1)
        wait_scatter(0)
        wait_idx(1)
        plsc.subcore_barrier()
        pltpu.sync_copy(agg_sh.at[pl.ds(base_row, RPT)],
                        out_hbm.at[cid, pl.ds(base_row, RPT)])

        @pl.when(sid == NS - 1)
        def _():
            pltpu.sync_copy(agg_sh.at[pl.ds(NS * RPT, TAIL)],
                            out_hbm.at[cid, pl.ds(NS * RPT, TAIL)])

    return k(src, dst, a_tab, b_tab)


def _tail_body(agg_ref, t0_ref, wn_ref, bn_ref, w2_ref, b2_ref,
               wl_ref, bl_ref, rm_ref, rv_ref, g_ref, bb_ref,
               w4_ref, b4_ref, out_ref):
    hi = jax.lax.Precision.HIGHEST
    agg = agg_ref[0] + agg_ref[1]
    h = jnp.maximum(
        jnp.dot(agg, wn_ref[...], precision=hi,
                preferred_element_type=jnp.float32) + bn_ref[...], 0.0)
    t = t0_ref[...] + jnp.dot(h, w2_ref[...], precision=hi,
                              preferred_element_type=jnp.float32) + b2_ref[...]
    for j in range(4):
        z = jnp.dot(t, wl_ref[j], precision=hi,
                    preferred_element_type=jnp.float32) + bl_ref[j]
        scale = jax.lax.rsqrt(rv_ref[j] + 1e-5) * g_ref[j]
        t = jnp.maximum((z - rm_ref[j]) * scale + bb_ref[j], 0.0)
    out_ref[...] = jnp.dot(t, w4_ref[...], precision=hi,
                           preferred_element_type=jnp.float32) + b4_ref[...]


def _tail(aggp, t0, wn_t, bn, w2_t, b2, wl_t, bl, rm, rv, g, bb, w4_t, b4):
    row = pl.BlockSpec((BM, D), lambda i: (i, 0))
    aggspec = pl.BlockSpec((NC, BM, D), lambda i: (0, i, 0))
    wspec = pl.BlockSpec((D, D), lambda i: (0, 0))
    bspec = pl.BlockSpec((1, D), lambda i: (0, 0))
    wlspec = pl.BlockSpec((4, D, D), lambda i: (0, 0, 0))
    blspec = pl.BlockSpec((4, 1, D), lambda i: (0, 0, 0))
    return pl.pallas_call(
        _tail_body,
        grid=(GRID,),
        in_specs=[aggspec, row, wspec, bspec, wspec, bspec,
                  wlspec, blspec, blspec, blspec, blspec, blspec,
                  wspec, bspec],
        out_specs=row,
        out_shape=jax.ShapeDtypeStruct((N, D), jnp.float32),
    )(aggp, t0, wn_t, bn.reshape(1, D), w2_t, b2.reshape(1, D),
      wl_t, bl.reshape(4, 1, D), rm.reshape(4, 1, D), rv.reshape(4, 1, D),
      g.reshape(4, 1, D), bb.reshape(4, 1, D), w4_t, b4.reshape(1, D))


def kernel(L_x_, L_edge_index_, L_self_modules_edge_lin_parameters_weight_, L_self_modules_edge_lin_parameters_bias_, L_self_modules_cat_lin1_parameters_weight_, L_self_modules_cat_lin1_parameters_bias_, L_self_modules_cat_lin2_parameters_weight_, L_self_modules_cat_lin2_parameters_bias_, L_self_modules_node_mlp_modules_lins_modules_0_parameters_weight_, L_self_modules_node_mlp_modules_lins_modules_0_parameters_bias_, L_self_modules_final_mlp_modules_lins_modules_0_parameters_weight_, L_self_modules_final_mlp_modules_lins_modules_0_parameters_bias_, L_self_modules_final_mlp_modules_lins_modules_1_parameters_weight_, L_self_modules_final_mlp_modules_lins_modules_1_parameters_bias_, L_self_modules_final_mlp_modules_lins_modules_2_parameters_weight_, L_self_modules_final_mlp_modules_lins_modules_2_parameters_bias_, L_self_modules_final_mlp_modules_lins_modules_3_parameters_weight_, L_self_modules_final_mlp_modules_lins_modules_3_parameters_bias_, L_self_modules_final_mlp_modules_lins_modules_4_parameters_weight_, L_self_modules_final_mlp_modules_lins_modules_4_parameters_bias_, L_self_modules_final_mlp_modules_norms_modules_0_modules_module_buffers_running_mean_, L_self_modules_final_mlp_modules_norms_modules_0_modules_module_buffers_running_var_, L_self_modules_final_mlp_modules_norms_modules_0_modules_module_parameters_weight_, L_self_modules_final_mlp_modules_norms_modules_0_modules_module_parameters_bias_, L_self_modules_final_mlp_modules_norms_modules_1_modules_module_buffers_running_mean_, L_self_modules_final_mlp_modules_norms_modules_1_modules_module_buffers_running_var_, L_self_modules_final_mlp_modules_norms_modules_1_modules_module_parameters_weight_, L_self_modules_final_mlp_modules_norms_modules_1_modules_module_parameters_bias_, L_self_modules_final_mlp_modules_norms_modules_2_modules_module_buffers_running_mean_, L_self_modules_final_mlp_modules_norms_modules_2_modules_module_buffers_running_var_, L_self_modules_final_mlp_modules_norms_modules_2_modules_module_parameters_weight_, L_self_modules_final_mlp_modules_norms_modules_2_modules_module_parameters_bias_, L_self_modules_final_mlp_modules_norms_modules_3_modules_module_buffers_running_mean_, L_self_modules_final_mlp_modules_norms_modules_3_modules_module_buffers_running_var_, L_self_modules_final_mlp_modules_norms_modules_3_modules_module_parameters_weight_, L_self_modules_final_mlp_modules_norms_modules_3_modules_module_parameters_bias_):
    x = L_x_
    ei = L_edge_index_
    src = ei[0].astype(jnp.int32)
    dst = ei[1].astype(jnp.int32)
    we = L_self_modules_edge_lin_parameters_weight_
    be = L_self_modules_edge_lin_parameters_bias_

    a_tab, b_tab, t0 = _pre(
        x, we[:, :D].T, we[:, D:].T, be,
        L_self_modules_cat_lin1_parameters_weight_.T,
        L_self_modules_cat_lin1_parameters_bias_)

    aggp = _edge_agg(src, dst, a_tab, b_tab)

    wl_t = jnp.stack([
        L_self_modules_final_mlp_modules_lins_modules_0_parameters_weight_.T,
        L_self_modules_final_mlp_modules_lins_modules_1_parameters_weight_.T,
        L_self_modules_final_mlp_modules_lins_modules_2_parameters_weight_.T,
        L_self_modules_final_mlp_modules_lins_modules_3_parameters_weight_.T,
    ])
    bl = jnp.stack([
        L_self_modules_final_mlp_modules_lins_modules_0_parameters_bias_,
        L_self_modules_final_mlp_modules_lins_modules_1_parameters_bias_,
        L_self_modules_final_mlp_modules_lins_modules_2_parameters_bias_,
        L_self_modules_final_mlp_modules_lins_modules_3_parameters_bias_,
    ])
    rm = jnp.stack([
        L_self_modules_final_mlp_modules_norms_modules_0_modules_module_buffers_running_mean_,
        L_self_modules_final_mlp_modules_norms_modules_1_modules_module_buffers_running_mean_,
        L_self_modules_final_mlp_modules_norms_modules_2_modules_module_buffers_running_mean_,
        L_self_modules_final_mlp_modules_norms_modules_3_modules_module_buffers_running_mean_,
    ])
    rv = jnp.stack([
        L_self_modules_final_mlp_modules_norms_modules_0_modules_module_buffers_running_var_,
        L_self_modules_final_mlp_modules_norms_modules_1_modules_module_buffers_running_var_,
        L_self_modules_final_mlp_modules_norms_modules_2_modules_module_buffers_running_var_,
        L_self_modules_final_mlp_modules_norms_modules_3_modules_module_buffers_running_var_,
    ])
    g = jnp.stack([
        L_self_modules_final_mlp_modules_norms_modules_0_modules_module_parameters_weight_,
        L_self_modules_final_mlp_modules_norms_modules_1_modules_module_parameters_weight_,
        L_self_modules_final_mlp_modules_norms_modules_2_modules_module_parameters_weight_,
        L_self_modules_final_mlp_modules_norms_modules_3_modules_module_parameters_weight_,
    ])
    bb = jnp.stack([
        L_self_modules_final_mlp_modules_norms_modules_0_modules_module_parameters_bias_,
        L_self_modules_final_mlp_modules_norms_modules_1_modules_module_parameters_bias_,
        L_self_modules_final_mlp_modules_norms_modules_2_modules_module_parameters_bias_,
        L_self_modules_final_mlp_modules_norms_modules_3_modules_module_parameters_bias_,
    ])

    return _tail(
        aggp, t0,
        L_self_modules_node_mlp_modules_lins_modules_0_parameters_weight_.T,
        L_self_modules_node_mlp_modules_lins_modules_0_parameters_bias_,
        L_self_modules_cat_lin2_parameters_weight_.T,
        L_self_modules_cat_lin2_parameters_bias_,
        wl_t, bl, rm, rv, g, bb,
        L_self_modules_final_mlp_modules_lins_modules_4_parameters_weight_.T,
        L_self_modules_final_mlp_modules_lins_modules_4_parameters_bias_)


# speedup vs baseline: 8.5274x; 1.3309x over previous
import functools

import jax
import jax.numpy as jnp
from jax import lax
from jax.experimental import pallas as pl
from jax.experimental.pallas import tpu as pltpu
from jax.experimental.pallas import tpu_sc as plsc

N = 10000
E = 320000
D = 128

NC = 2
NS = 16
NW = NC * NS
EPW = E // NW
K = 80
NCHUNK = EPW // K
RPT = 624
TAIL = N - NS * RPT

BM = 1000
GRID = N // BM


def _pre_body(x_ref, wes_ref, wed_ref, be_ref, w1_ref, b1_ref,
              a_ref, b_ref, t0_ref):
    x = x_ref[...]
    a_ref[...] = jnp.dot(x, wes_ref[...],
                         preferred_element_type=jnp.float32)
    b_ref[...] = jnp.dot(x, wed_ref[...],
                         preferred_element_type=jnp.float32) + be_ref[...]
    t0_ref[...] = jnp.dot(x, w1_ref[...],
                          preferred_element_type=jnp.float32) + b1_ref[...]


def _pre(x, wes_t, wed_t, be, w1_t, b1):
    row = pl.BlockSpec((BM, D), lambda i: (i, 0))
    wspec = pl.BlockSpec((D, D), lambda i: (0, 0))
    bspec = pl.BlockSpec((1, D), lambda i: (0, 0))
    return pl.pallas_call(
        _pre_body,
        grid=(GRID,),
        in_specs=[row, wspec, wspec, bspec, wspec, bspec],
        out_specs=[row, row, row],
        out_shape=[jax.ShapeDtypeStruct((N, D), jnp.float32)] * 3,
    )(x, wes_t, wed_t, be.reshape(1, D), w1_t, b1.reshape(1, D))


def _edge_agg(src, dst, a_tab, b_tab):
    mesh = plsc.VectorSubcoreMesh(core_axis_name="c", subcore_axis_name="s")

    @functools.partial(
        pl.kernel,
        mesh=mesh,
        out_type=jax.ShapeDtypeStruct((NC, N, D), jnp.float32),
        scratch_types=[
            pltpu.VMEM((K,), jnp.int32),
            pltpu.VMEM((K,), jnp.int32),
            pltpu.VMEM((K,), jnp.int32),
            pltpu.VMEM((K,), jnp.int32),
            pltpu.VMEM((K,), jnp.int32),
            pltpu.VMEM((K,), jnp.int32),
            pltpu.VMEM((K, D), jnp.float32),
            pltpu.VMEM((K, D), jnp.float32),
            pltpu.VMEM((K, D), jnp.float32),
            pltpu.VMEM((K, D), jnp.float32),
            pltpu.VMEM_SHARED((N, D), jnp.float32),
            pltpu.SemaphoreType.DMA,
            pltpu.SemaphoreType.DMA,
            pltpu.SemaphoreType.DMA,
            pltpu.SemaphoreType.DMA,
            pltpu.SemaphoreType.DMA,
            pltpu.SemaphoreType.DMA,
        ],
    )
    def k(src_hbm, dst_hbm, a_hbm, b_hbm, out_hbm,
          si0, si1, di0, di1, sd0, sd1, a0, a1, b0, b1, agg_sh,
          gsem0, gsem1, isem0, isem1, ssem0, ssem1):
        cid = lax.axis_index("c")
        sid = lax.axis_index("s")
        wid = cid * NS + sid
        sibuf = (si0, si1)
        dibuf = (di0, di1)
        sdbuf = (sd0, sd1)
        abuf = (a0, a1)
        bbuf = (b0, b1)
        gsem = (gsem0, gsem1)
        isem = (isem0, isem1)
        ssem = (ssem0, ssem1)
        ebase = wid * EPW

        zero = jnp.zeros((16,), jnp.float32)

        def zrow(j, carry):
            for r in range(8):
                b0[j, pl.ds(r * 16, 16)] = zero
            return carry

        lax.fori_loop(0, K, zrow, 0)

        base_row = sid * RPT
        for r in range(RPT // K):
            pltpu.sync_copy(b0, agg_sh.at[pl.ds(base_row + r * K, K)])
        if RPT % K:
            pltpu.sync_copy(
                b0.at[pl.ds(0, RPT % K)],
                agg_sh.at[pl.ds(base_row + (RPT // K) * K, RPT % K)])

        @pl.when(sid == NS - 1)
        def _():
            pltpu.sync_copy(b0.at[pl.ds(0, TAIL)],
                            agg_sh.at[pl.ds(NS * RPT, TAIL)])

        plsc.subcore_barrier()

        def fetch_idx(c, s):
            off = ebase + c * K
            pltpu.async_copy(src_hbm.at[pl.ds(off, K)], sibuf[s], isem[s])
            pltpu.async_copy(dst_hbm.at[pl.ds(off, K)], dibuf[s], isem[s])

        def wait_idx(s):
            pltpu.make_async_copy(src_hbm.at[pl.ds(0, K)], sibuf[s],
                                  isem[s]).wait()
            pltpu.make_async_copy(dst_hbm.at[pl.ds(0, K)], dibuf[s],
                                  isem[s]).wait()

        def gathers(s):
            pltpu.async_copy(a_hbm.at[sibuf[s]], abuf[s], gsem[s])
            pltpu.async_copy(b_hbm.at[dibuf[s]], bbuf[s], gsem[s])

        def wait_gathers(s):
            pltpu.make_async_copy(a_hbm.at[sibuf[s]], abuf[s],
                                  gsem[s]).wait()
            pltpu.make_async_copy(b_hbm.at[dibuf[s]], bbuf[s],
                                  gsem[s]).wait()

        def compute(s):
            wait_gathers(s)
            a_v, b_v = abuf[s], bbuf[s]

            def row(j, cc):
                for u in range(2):
                    for r in range(8):
                        sl = pl.ds(r * 16, 16)
                        b_v[2 * j + u, sl] = jnp.maximum(
                            a_v[2 * j + u, sl] + b_v[2 * j + u, sl], 0.0)
                return cc

            lax.fori_loop(0, K // 2, row, 0)
            for r in range(K // 16):
                sl = pl.ds(r * 16, 16)
                sdbuf[s][sl] = dibuf[s][sl]

        def scatter(s):
            pltpu.async_copy(bbuf[s], agg_sh.at[sdbuf[s]], ssem[s],
                             add=True)

        def wait_scatter(s):
            pltpu.make_async_copy(bbuf[s], agg_sh.at[sdbuf[s]],
                                  ssem[s]).wait()

        fetch_idx(0, 0)
        wait_idx(0)
        gathers(0)
        fetch_idx(1, 1)

        def pair(i, carry):
            c = 2 * i
            wait_idx(1)

            @pl.when(i > 0)
            def _():
                wait_scatter(1)

            gathers(1)
            compute(0)
            scatter(0)
            fetch_idx(c + 2, 0)
            compute(1)
            scatter(1)
            fetch_idx(jnp.minimum(c + 3, NCHUNK - 1), 1)
            wait_idx(0)
            wait_scatter(0)
            gathers(0)
            return carry

        lax.fori_loop(0, (NCHUNK - 1) // 2, pair, 0)
        compute(0)
        scatter(0)
        wait_scatter(1)
        wait_scatter(0)
        wait_idx(1)
        plsc.subcore_barrier()
        pltpu.sync_copy(agg_sh.at[pl.ds(base_row, RPT)],
                        out_hbm.at[cid, pl.ds(base_row, RPT)])

        @pl.when(sid == NS - 1)
        def _():
            pltpu.sync_copy(agg_sh.at[pl.ds(NS * RPT, TAIL)],
                            out_hbm.at[cid, pl.ds(NS * RPT, TAIL)])

    return k(src, dst, a_tab, b_tab)


def _tail_body(agg_ref, t0_ref, wn_ref, bn_ref, w2_ref, b2_ref,
               wl_ref, bl_ref, rm_ref, rv_ref, g_ref, bb_ref,
               w4_ref, b4_ref, out_ref):
    agg = agg_ref[0] + agg_ref[1]
    h = jnp.maximum(
        jnp.dot(agg, wn_ref[...],
                preferred_element_type=jnp.float32) + bn_ref[...], 0.0)
    t = t0_ref[...] + jnp.dot(h, w2_ref[...],
                              preferred_element_type=jnp.float32) + b2_ref[...]
    for j in range(4):
        z = jnp.dot(t, wl_ref[j],
                    preferred_element_type=jnp.float32) + bl_ref[j]
        scale = jax.lax.rsqrt(rv_ref[j] + 1e-5) * g_ref[j]
        t = jnp.maximum((z - rm_ref[j]) * scale + bb_ref[j], 0.0)
    out_ref[...] = jnp.dot(t, w4_ref[...],
                           preferred_element_type=jnp.float32) + b4_ref[...]


def _tail(aggp, t0, wn_t, bn, w2_t, b2, wl_t, bl, rm, rv, g, bb, w4_t, b4):
    row = pl.BlockSpec((BM, D), lambda i: (i, 0))
    aggspec = pl.BlockSpec((NC, BM, D), lambda i: (0, i, 0))
    wspec = pl.BlockSpec((D, D), lambda i: (0, 0))
    bspec = pl.BlockSpec((1, D), lambda i: (0, 0))
    wlspec = pl.BlockSpec((4, D, D), lambda i: (0, 0, 0))
    blspec = pl.BlockSpec((4, 1, D), lambda i: (0, 0, 0))
    return pl.pallas_call(
        _tail_body,
        grid=(GRID,),
        in_specs=[aggspec, row, wspec, bspec, wspec, bspec,
                  wlspec, blspec, blspec, blspec, blspec, blspec,
                  wspec, bspec],
        out_specs=row,
        out_shape=jax.ShapeDtypeStruct((N, D), jnp.float32),
    )(aggp, t0, wn_t, bn.reshape(1, D), w2_t, b2.reshape(1, D),
      wl_t, bl.reshape(4, 1, D), rm.reshape(4, 1, D), rv.reshape(4, 1, D),
      g.reshape(4, 1, D), bb.reshape(4, 1, D), w4_t, b4.reshape(1, D))


def kernel(L_x_, L_edge_index_, L_self_modules_edge_lin_parameters_weight_, L_self_modules_edge_lin_parameters_bias_, L_self_modules_cat_lin1_parameters_weight_, L_self_modules_cat_lin1_parameters_bias_, L_self_modules_cat_lin2_parameters_weight_, L_self_modules_cat_lin2_parameters_bias_, L_self_modules_node_mlp_modules_lins_modules_0_parameters_weight_, L_self_modules_node_mlp_modules_lins_modules_0_parameters_bias_, L_self_modules_final_mlp_modules_lins_modules_0_parameters_weight_, L_self_modules_final_mlp_modules_lins_modules_0_parameters_bias_, L_self_modules_final_mlp_modules_lins_modules_1_parameters_weight_, L_self_modules_final_mlp_modules_lins_modules_1_parameters_bias_, L_self_modules_final_mlp_modules_lins_modules_2_parameters_weight_, L_self_modules_final_mlp_modules_lins_modules_2_parameters_bias_, L_self_modules_final_mlp_modules_lins_modules_3_parameters_weight_, L_self_modules_final_mlp_modules_lins_modules_3_parameters_bias_, L_self_modules_final_mlp_modules_lins_modules_4_parameters_weight_, L_self_modules_final_mlp_modules_lins_modules_4_parameters_bias_, L_self_modules_final_mlp_modules_norms_modules_0_modules_module_buffers_running_mean_, L_self_modules_final_mlp_modules_norms_modules_0_modules_module_buffers_running_var_, L_self_modules_final_mlp_modules_norms_modules_0_modules_module_parameters_weight_, L_self_modules_final_mlp_modules_norms_modules_0_modules_module_parameters_bias_, L_self_modules_final_mlp_modules_norms_modules_1_modules_module_buffers_running_mean_, L_self_modules_final_mlp_modules_norms_modules_1_modules_module_buffers_running_var_, L_self_modules_final_mlp_modules_norms_modules_1_modules_module_parameters_weight_, L_self_modules_final_mlp_modules_norms_modules_1_modules_module_parameters_bias_, L_self_modules_final_mlp_modules_norms_modules_2_modules_module_buffers_running_mean_, L_self_modules_final_mlp_modules_norms_modules_2_modules_module_buffers_running_var_, L_self_modules_final_mlp_modules_norms_modules_2_modules_module_parameters_weight_, L_self_modules_final_mlp_modules_norms_modules_2_modules_module_parameters_bias_, L_self_modules_final_mlp_modules_norms_modules_3_modules_module_buffers_running_mean_, L_self_modules_final_mlp_modules_norms_modules_3_modules_module_buffers_running_var_, L_self_modules_final_mlp_modules_norms_modules_3_modules_module_parameters_weight_, L_self_modules_final_mlp_modules_norms_modules_3_modules_module_parameters_bias_):
    x = L_x_
    ei = L_edge_index_
    src = ei[0].astype(jnp.int32)
    dst = ei[1].astype(jnp.int32)
    we = L_self_modules_edge_lin_parameters_weight_
    be = L_self_modules_edge_lin_parameters_bias_

    a_tab, b_tab, t0 = _pre(
        x, we[:, :D].T, we[:, D:].T, be,
        L_self_modules_cat_lin1_parameters_weight_.T,
        L_self_modules_cat_lin1_parameters_bias_)

    aggp = _edge_agg(src, dst, a_tab, b_tab)

    wl_t = jnp.stack([
        L_self_modules_final_mlp_modules_lins_modules_0_parameters_weight_.T,
        L_self_modules_final_mlp_modules_lins_modules_1_parameters_weight_.T,
        L_self_modules_final_mlp_modules_lins_modules_2_parameters_weight_.T,
        L_self_modules_final_mlp_modules_lins_modules_3_parameters_weight_.T,
    ])
    bl = jnp.stack([
        L_self_modules_final_mlp_modules_lins_modules_0_parameters_bias_,
        L_self_modules_final_mlp_modules_lins_modules_1_parameters_bias_,
        L_self_modules_final_mlp_modules_lins_modules_2_parameters_bias_,
        L_self_modules_final_mlp_modules_lins_modules_3_parameters_bias_,
    ])
    rm = jnp.stack([
        L_self_modules_final_mlp_modules_norms_modules_0_modules_module_buffers_running_mean_,
        L_self_modules_final_mlp_modules_norms_modules_1_modules_module_buffers_running_mean_,
        L_self_modules_final_mlp_modules_norms_modules_2_modules_module_buffers_running_mean_,
        L_self_modules_final_mlp_modules_norms_modules_3_modules_module_buffers_running_mean_,
    ])
    rv = jnp.stack([
        L_self_modules_final_mlp_modules_norms_modules_0_modules_module_buffers_running_var_,
        L_self_modules_final_mlp_modules_norms_modules_1_modules_module_buffers_running_var_,
        L_self_modules_final_mlp_modules_norms_modules_2_modules_module_buffers_running_var_,
        L_self_modules_final_mlp_modules_norms_modules_3_modules_module_buffers_running_var_,
    ])
    g = jnp.stack([
        L_self_modules_final_mlp_modules_norms_modules_0_modules_module_parameters_weight_,
        L_self_modules_final_mlp_modules_norms_modules_1_modules_module_parameters_weight_,
        L_self_modules_final_mlp_modules_norms_modules_2_modules_module_parameters_weight_,
        L_self_modules_final_mlp_modules_norms_modules_3_modules_module_parameters_weight_,
    ])
    bb = jnp.stack([
        L_self_modules_final_mlp_modules_norms_modules_0_modules_module_parameters_bias_,
        L_self_modules_final_mlp_modules_norms_modules_1_modules_module_parameters_bias_,
        L_self_modules_final_mlp_modules_norms_modules_2_modules_module_parameters_bias_,
        L_self_modules_final_mlp_modules_norms_modules_3_modules_module_parameters_bias_,
    ])

    return _tail(
        aggp, t0,
        L_self_modules_node_mlp_modules_lins_modules_0_parameters_weight_.T,
        L_self_modules_node_mlp_modules_lins_modules_0_parameters_bias_,
        L_self_modules_cat_lin2_parameters_weight_.T,
        L_self_modules_cat_lin2_parameters_bias_,
        wl_t, bl, rm, rv, g, bb,
        L_self_modules_final_mlp_modules_lins_modules_4_parameters_weight_.T,
        L_self_modules_final_mlp_modules_lins_modules_4_parameters_bias_)
